# baseline (speedup 1.0000x reference)
import jax
import jax.numpy as jnp
from jax import lax
from jax.experimental import pallas as pl
from jax.experimental.pallas import tpu as pltpu
from jax.experimental.pallas import tpu_sc as plsc

_X_MIN, _Y_MIN, _Z_MIN = 0.0, -6.4, -2.0
_VOX = 0.4
_X_MAX_I, _Y_MAX_I, _Z_MAX_I = 179.0, 31.0, 19.0

_N = 2_000_000
_C = 4_000
_NCHUNK = _N // _C
_NW = 32
_JMAX = -(-_NCHUNK // _NW)
_G = _C // 16


def _voxel_idx(v, vmin, vmax_f):
    t = (v - vmin) / _VOX
    t = jnp.minimum(jnp.maximum(t, 0.0), vmax_f)
    return t.astype(jnp.int32)


def _sc_body(rdr_hbm, b_hbm, out_hbm, pts_v, b_v, out_v):
    wid = lax.axis_index("s") * 2 + lax.axis_index("c")
    it5 = lax.iota(jnp.int32, 16) * 5
    it4 = lax.iota(jnp.int32, 16) * 4

    for j in range(_JMAX):
        cid = wid + _NW * j

        @pl.when(cid < _NCHUNK)
        def _():
            base = cid * _C
            pltpu.sync_copy(rdr_hbm.at[pl.ds(base * 5, _C * 5)], pts_v)
            pltpu.sync_copy(b_hbm.at[pl.ds(base, _C)], b_v)

            def grp(i, carry):
                src = it5 + i * 80
                dst = it4 + i * 64
                xv = plsc.load_gather(pts_v, [src])
                yv = plsc.load_gather(pts_v, [src + 1])
                zv = plsc.load_gather(pts_v, [src + 2])
                bv = b_v[pl.ds(i * 16, 16)]
                xi = _voxel_idx(xv, _X_MIN, _X_MAX_I)
                yi = _voxel_idx(yv, _Y_MIN, _Y_MAX_I)
                zi = _voxel_idx(zv, _Z_MIN, _Z_MAX_I)
                plsc.store_scatter(out_v, [dst], bv)
                plsc.store_scatter(out_v, [dst + 1], zi)
                plsc.store_scatter(out_v, [dst + 2], yi)
                plsc.store_scatter(out_v, [dst + 3], xi)
                return carry

            lax.fori_loop(0, _G, grp, 0)
            pltpu.sync_copy(out_v, out_hbm.at[pl.ds(base * 4, _C * 4)])


def kernel(rdr_filt_srt, pts_batch_indices_rdr_filt_srt):
    mesh = plsc.VectorSubcoreMesh(core_axis_name="c", subcore_axis_name="s")
    f = pl.kernel(
        _sc_body,
        mesh=mesh,
        compiler_params=pltpu.CompilerParams(needs_layout_passes=False),
        out_type=jax.ShapeDtypeStruct((_N * 4,), jnp.int32),
        scratch_types=[
            pltpu.VMEM((_C * 5,), jnp.float32),
            pltpu.VMEM((_C,), jnp.int32),
            pltpu.VMEM((_C * 4,), jnp.int32),
        ],
    )
    sp_flat = f(
        rdr_filt_srt.reshape(-1),
        pts_batch_indices_rdr_filt_srt.astype(jnp.int32),
    )
    return rdr_filt_srt, sp_flat.reshape(_N, 4)


# speedup vs baseline: 1.0241x; 1.0241x over previous
import jax
import jax.numpy as jnp
from jax import lax
from jax.experimental import pallas as pl
from jax.experimental.pallas import tpu as pltpu
from jax.experimental.pallas import tpu_sc as plsc

_X_MIN, _Y_MIN, _Z_MIN = 0.0, -6.4, -2.0
_VOX = 0.4
_X_MAX_I, _Y_MAX_I, _Z_MAX_I = 179.0, 31.0, 19.0

_N = 2_000_000
_C = 4_000
_NCHUNK = _N // _C
_NW = 32
_JMAX = -(-_NCHUNK // _NW)
_G = _C // 16


def _voxel_idx(v, vmin, vmax_f):
    t = (v - vmin) / _VOX
    t = jnp.minimum(jnp.maximum(t, 0.0), vmax_f)
    return t.astype(jnp.int32)


def _sc_body(rdr_hbm, b_hbm, out_hbm,
             pts0, pts1, b0, b1, o0, o1,
             sin0, sin1, sout0, sout1):
    wid = lax.axis_index("s") * 2 + lax.axis_index("c")
    pts = (pts0, pts1)
    bbuf = (b0, b1)
    obuf = (o0, o1)
    sin = (sin0, sin1)
    sout = (sout0, sout1)
    it5 = lax.iota(jnp.int32, 16) * 5
    it4 = lax.iota(jnp.int32, 16) * 4

    def base_of(j):
        return jnp.minimum(wid + _NW * j, _NCHUNK - 1) * _C

    def start_in(j):
        k = j % 2
        base = base_of(j)
        c1 = pltpu.async_copy(rdr_hbm.at[pl.ds(base * 5, _C * 5)], pts[k], sin[k])
        c2 = pltpu.async_copy(b_hbm.at[pl.ds(base, _C)], bbuf[k], sin[k])
        return (c1, c2)

    h_in = [None, None]
    h_out = [None, None]
    h_in[0] = start_in(0)

    for j in range(_JMAX):
        k = j % 2
        if j + 1 < _JMAX:
            h_in[(j + 1) % 2] = start_in(j + 1)
        h_in[k][0].wait()
        h_in[k][1].wait()
        if h_out[k] is not None:
            h_out[k].wait()

        pts_v, b_v, out_v = pts[k], bbuf[k], obuf[k]

        @plsc.parallel_loop(0, _G, unroll=4)
        def grp(i):
            src = it5 + i * 80
            dst = it4 + i * 64
            xv = plsc.load_gather(pts_v, [src])
            yv = plsc.load_gather(pts_v, [src + 1])
            zv = plsc.load_gather(pts_v, [src + 2])
            bv = b_v[pl.ds(i * 16, 16)]
            xi = _voxel_idx(xv, _X_MIN, _X_MAX_I)
            yi = _voxel_idx(yv, _Y_MIN, _Y_MAX_I)
            zi = _voxel_idx(zv, _Z_MIN, _Z_MAX_I)
            plsc.store_scatter(out_v, [dst], bv)
            plsc.store_scatter(out_v, [dst + 1], zi)
            plsc.store_scatter(out_v, [dst + 2], yi)
            plsc.store_scatter(out_v, [dst + 3], xi)

        h_out[k] = pltpu.async_copy(
            out_v, out_hbm.at[pl.ds(base_of(j) * 4, _C * 4)], sout[k]
        )

    h_out[0].wait()
    h_out[1].wait()


def kernel(rdr_filt_srt, pts_batch_indices_rdr_filt_srt):
    mesh = plsc.VectorSubcoreMesh(core_axis_name="c", subcore_axis_name="s")
    f = pl.kernel(
        _sc_body,
        mesh=mesh,
        compiler_params=pltpu.CompilerParams(needs_layout_passes=False),
        out_type=jax.ShapeDtypeStruct((_N * 4,), jnp.int32),
        scratch_types=[
            pltpu.VMEM((_C * 5,), jnp.float32),
            pltpu.VMEM((_C * 5,), jnp.float32),
            pltpu.VMEM((_C,), jnp.int32),
            pltpu.VMEM((_C,), jnp.int32),
            pltpu.VMEM((_C * 4,), jnp.int32),
            pltpu.VMEM((_C * 4,), jnp.int32),
            pltpu.SemaphoreType.DMA,
            pltpu.SemaphoreType.DMA,
            pltpu.SemaphoreType.DMA,
            pltpu.SemaphoreType.DMA,
        ],
    )
    sp_flat = f(
        rdr_filt_srt.reshape(-1),
        pts_batch_indices_rdr_filt_srt.astype(jnp.int32),
    )
    return rdr_filt_srt, sp_flat.reshape(_N, 4)
